# baseline (speedup 1.0000x reference)
import jax
import jax.numpy as jnp
from jax import lax
from jax.experimental import pallas as pl
from jax.experimental.pallas import tpu as pltpu

H = 2048
E = 16
NF = 16384
FE = NF // E

BF = 256
G = FE // BF


def _body(act_ref, eb_ref, rb_ref, router_ref, enc_hbm, dec_hbm, out_ref,
          acc_ref):
    xr = act_ref[...] - rb_ref[...]
    logits = jnp.dot(xr, router_ref[...],
                     preferred_element_type=jnp.float32)
    m = jnp.max(logits)
    maxp = 1.0 / jnp.sum(jnp.exp(logits - m))
    iota = lax.broadcasted_iota(jnp.int32, (1, E), 1)
    idx = jnp.min(jnp.where(logits == m, iota, E))

    x = act_ref[...] - eb_ref[...]
    acc_ref[...] = jnp.zeros_like(acc_ref)

    def inner(enc_blk, dec_blk):
        f = jnp.dot(x, enc_blk[0],
                    preferred_element_type=jnp.float32)
        f = jnp.maximum(f, 0.0)
        acc_ref[...] += jnp.dot(f, dec_blk[0],
                                preferred_element_type=jnp.float32)

    pltpu.emit_pipeline(
        inner,
        grid=(G,),
        in_specs=[
            pl.BlockSpec((1, H, BF), lambda i: (idx, 0, i)),
            pl.BlockSpec((1, BF, H), lambda i: (idx, i, 0)),
        ],
    )(enc_hbm, dec_hbm)

    out_ref[...] = acc_ref[...] * maxp + eb_ref[...]


def kernel(activations, enc, dec, expert_b, router_b, router):
    act2 = activations.reshape(1, H)
    rb2 = router_b.reshape(1, H)
    eb2 = expert_b.reshape(1, H)

    out = pl.pallas_call(
        _body,
        in_specs=[
            pl.BlockSpec(memory_space=pltpu.VMEM),
            pl.BlockSpec(memory_space=pltpu.VMEM),
            pl.BlockSpec(memory_space=pltpu.VMEM),
            pl.BlockSpec(memory_space=pltpu.VMEM),
            pl.BlockSpec(memory_space=pl.ANY),
            pl.BlockSpec(memory_space=pl.ANY),
        ],
        out_specs=pl.BlockSpec(memory_space=pltpu.VMEM),
        out_shape=jax.ShapeDtypeStruct((1, H), jnp.float32),
        scratch_shapes=[
            pltpu.VMEM((1, H), jnp.float32),
        ],
    )(act2, eb2, rb2, router, enc, dec)

    return out.reshape(H)
